# baseline (speedup 1.0000x reference)
import functools

import jax
import jax.numpy as jnp
from jax import lax
from jax.experimental import pallas as pl
from jax.experimental.pallas import tpu as pltpu
from jax.experimental.pallas import tpu_sc as plsc

N = 10000
E = 320000
D = 128
R = 16
NB = 4

NC = 2
NS = 16
NWORK = NC * NS
CHE = 128
CB = 128
NSEG = 5
RW = 79
NP = (RW + 1) // 2
EP_ROWS = NWORK * RW
STRIPE = 624
BN = 1000


def _tc_proj1_body(coeff_ref, basis_ref, x_ref, out_ref, w_ref):
    @pl.when(pl.program_id(0) == 0)
    def _():
        for r in range(R):
            acc = coeff_ref[r, 0] * basis_ref[0]
            for b in range(1, NB):
                acc = acc + coeff_ref[r, b] * basis_ref[b]
            w_ref[:, r * D:(r + 1) * D] = acc

    out_ref[...] = jnp.dot(x_ref[...], w_ref[...],
                           preferred_element_type=jnp.float32)


def _tc_proj2_body(coeff_ref, basis_ref, p_ref, bias_ref, out_ref, w_ref):
    @pl.when(pl.program_id(0) == 0)
    def _():
        for r in range(R):
            acc = coeff_ref[r, 0] * basis_ref[0]
            for b in range(1, NB):
                acc = acc + coeff_ref[r, b] * basis_ref[b]
            w_ref[:, r * D:(r + 1) * D] = acc

    x = jnp.maximum(p_ref[0] + p_ref[1] + bias_ref[...], 0.0)
    out_ref[...] = jnp.dot(x, w_ref[...], preferred_element_type=jnp.float32)


def _tc_merge_body(p_ref, bias_ref, out_ref):
    out_ref[...] = p_ref[0] + p_ref[1] + bias_ref[...]


def _tc_pack_body(src_ref, et_ref, out_ref):
    out_ref[...] = src_ref[...] * R + et_ref[...]


def _pack_gidx(src2, et2):
    return pl.pallas_call(
        _tc_pack_body,
        out_shape=jax.ShapeDtypeStruct((EP_ROWS, CHE), jnp.int32),
    )(src2, et2)


def _proj1(coeff, basis, x):
    return pl.pallas_call(
        _tc_proj1_body,
        grid=(N // BN,),
        in_specs=[
            pl.BlockSpec(memory_space=pltpu.SMEM),
            pl.BlockSpec((NB, D, D), lambda i: (0, 0, 0)),
            pl.BlockSpec((BN, D), lambda i: (i, 0)),
        ],
        out_specs=pl.BlockSpec((BN, R * D), lambda i: (i, 0)),
        out_shape=jax.ShapeDtypeStruct((N, R * D), jnp.float32),
        scratch_shapes=[pltpu.VMEM((D, R * D), jnp.float32)],
    )(coeff, basis, x)


def _proj2(coeff, basis, parts, bias):
    return pl.pallas_call(
        _tc_proj2_body,
        grid=(N // BN,),
        in_specs=[
            pl.BlockSpec(memory_space=pltpu.SMEM),
            pl.BlockSpec((NB, D, D), lambda i: (0, 0, 0)),
            pl.BlockSpec((NC, BN, D), lambda i: (0, i, 0)),
            pl.BlockSpec((1, D), lambda i: (0, 0)),
        ],
        out_specs=pl.BlockSpec((BN, R * D), lambda i: (i, 0)),
        out_shape=jax.ShapeDtypeStruct((N, R * D), jnp.float32),
        scratch_shapes=[pltpu.VMEM((D, R * D), jnp.float32)],
    )(coeff, basis, parts, bias)


def _merge(parts, bias):
    return pl.pallas_call(
        _tc_merge_body,
        grid=(N // BN,),
        in_specs=[
            pl.BlockSpec((NC, BN, D), lambda i: (0, i, 0)),
            pl.BlockSpec((1, D), lambda i: (0, 0)),
        ],
        out_specs=pl.BlockSpec((BN, D), lambda i: (i, 0)),
        out_shape=jax.ShapeDtypeStruct((N, D), jnp.float32),
    )(parts, bias)


_SC_MESH = plsc.VectorSubcoreMesh(core_axis_name="c", subcore_axis_name="s")


@functools.partial(
    pl.kernel,
    out_type=jax.ShapeDtypeStruct((NC, N, D), jnp.float32),
    mesh=_SC_MESH,
    scratch_types=[
        pltpu.VMEM((2, 8, CHE), jnp.int32),
        pltpu.VMEM((2, 8, CHE), jnp.int32),
        pltpu.VMEM((2, 8, CHE), jnp.float32),
        pltpu.VMEM((RW + 1,), jnp.int32),
        pltpu.VMEM((CB,), jnp.int32),
        pltpu.VMEM((CB,), jnp.int32),
        pltpu.VMEM((CB, D), jnp.float32),
        pltpu.VMEM((CB, D), jnp.float32),
        pltpu.VMEM((16, D), jnp.float32),
        pltpu.VMEM_SHARED((N, D), jnp.float32),
        pltpu.SemaphoreType.DMA,
        pltpu.SemaphoreType.DMA,
        pltpu.SemaphoreType.DMA,
        pltpu.SemaphoreType.DMA,
        pltpu.SemaphoreType.DMA,
        pltpu.SemaphoreType.DMA,
    ],
)
def _sc_edge_pass(proj_hbm, gidx_hbm, dst_hbm, norm_hbm, out_hbm,
                  gidx_v, dst_v, norm_v, sidx, dst_a, dst_b,
                  rows_a, rows_b, zero_v, acc,
                  sem_ga, sem_gb, sem_sa, sem_sb, sem_e0, sem_e1):
    c = lax.axis_index("c")
    s = lax.axis_index("s")
    w = c * NS + s
    base = w * RW

    for j in range((RW + 1) // 16):
        sidx[pl.ds(j * 16, 16)] = jnp.minimum(
            lax.iota(jnp.int32, 16) + (base + j * 16), base + RW - 1)
    tail = lax.iota(jnp.int32, 16)
    sidx[pl.ds(64, 16)] = jnp.where(tail == 15, EP_ROWS - 1,
                                    tail + (base + 64))

    def _edge_fetch(grp, slot_id):
        sem_e = sem_e0 if slot_id == 0 else sem_e1
        off = pl.multiple_of(jnp.minimum(grp, 9) * 8, 8)
        sl = pl.ds(off, 8)
        pltpu.async_copy(gidx_hbm.at[sidx.at[sl]], gidx_v.at[slot_id], sem_e)
        pltpu.async_copy(dst_hbm.at[sidx.at[sl]], dst_v.at[slot_id], sem_e)
        pltpu.async_copy(norm_hbm.at[sidx.at[sl]], norm_v.at[slot_id], sem_e)

    def _edge_wait(slot_id):
        sem_e = sem_e0 if slot_id == 0 else sem_e1
        sl = pl.ds(0, 8)
        pltpu.make_async_copy(gidx_hbm.at[sidx.at[sl]], gidx_v.at[slot_id],
                              sem_e).wait()
        pltpu.make_async_copy(dst_hbm.at[sidx.at[sl]], dst_v.at[slot_id],
                              sem_e).wait()
        pltpu.make_async_copy(norm_hbm.at[sidx.at[sl]], norm_v.at[slot_id],
                              sem_e).wait()

    _edge_fetch(0, 0)

    zv = jnp.zeros((16,), jnp.float32)
    for i in range(16):
        for j in range(D // 16):
            zero_v[i, pl.ds(j * 16, 16)] = zv
    sbase = pl.multiple_of(s * STRIPE, 8)

    def _zero_blk(i, carry):
        off = pl.multiple_of(sbase + i * 16, 8)
        pltpu.sync_copy(zero_v, acc.at[pl.ds(off, 16)])
        return carry

    lax.fori_loop(0, STRIPE // 16, _zero_blk, 0)

    @pl.when(s == NS - 1)
    def _():
        pltpu.sync_copy(zero_v, acc.at[pl.ds(NS * STRIPE, 16)])

    plsc.subcore_barrier()

    _edge_wait(0)

    def _prep(slot, rl, dst_ch):
        for j in range(CB // 16):
            d = pl.ds(j * 16, 16)
            dst_ch[d] = dst_v[slot, rl, d]

    def _scale(slot, rl, rows_v):
        def _grp(q, inner):
            nv = norm_v[slot, rl, pl.ds(q * 16, 16)]
            for k in range(16):
                sv = jnp.full((16,), nv[k], jnp.float32)
                for j in range(D // 16):
                    sl2 = pl.ds(j * 16, 16)
                    e = q * 16 + k
                    rows_v[e, sl2] = rows_v[e, sl2] * sv
            return inner

        lax.fori_loop(0, CB // 16, _grp, 0)

    _prep(0, 0, dst_a)
    pltpu.async_copy(proj_hbm.at[gidx_v.at[0, 0]], rows_a, sem_ga)

    def _pair(p, carry):
        g = p // 4
        pmod = lax.rem(p, 4)
        slot = lax.rem(g, 2)
        rla = 2 * pmod
        rlb = rla + 1
        gn = (p + 1) // 4
        slot_n = lax.rem(gn, 2)
        rln = 2 * lax.rem(p + 1, 4)

        @pl.when(jnp.logical_and(pmod == 0, slot == 0))
        def _():
            _edge_fetch(g + 1, 1)

        @pl.when(jnp.logical_and(pmod == 0, slot == 1))
        def _():
            _edge_fetch(g + 1, 0)

        @pl.when(p > 0)
        def _():
            pltpu.make_async_copy(rows_b, acc.at[dst_b], sem_sb).wait()

        _prep(slot, rlb, dst_b)
        pltpu.async_copy(proj_hbm.at[gidx_v.at[slot, rlb]], rows_b, sem_gb)
        pltpu.make_async_copy(proj_hbm.at[gidx_v.at[slot, rla]], rows_a,
                              sem_ga).wait()
        _scale(slot, rla, rows_a)
        pltpu.async_copy(rows_a, acc.at[dst_a], sem_sa, add=True)
        pltpu.make_async_copy(proj_hbm.at[gidx_v.at[slot, rlb]], rows_b,
                              sem_gb).wait()
        _scale(slot, rlb, rows_b)
        pltpu.async_copy(rows_b, acc.at[dst_b], sem_sb, add=True)
        pltpu.make_async_copy(rows_a, acc.at[dst_a], sem_sa).wait()

        @pl.when(jnp.logical_and(pmod == 3, slot == 0))
        def _():
            _edge_wait(1)

        @pl.when(jnp.logical_and(pmod == 3, slot == 1))
        def _():
            _edge_wait(0)

        _prep(slot_n, rln, dst_a)
        pltpu.async_copy(proj_hbm.at[gidx_v.at[slot_n, rln]], rows_a, sem_ga)
        return carry

    lax.fori_loop(0, NP, _pair, 0)
    pltpu.make_async_copy(proj_hbm.at[gidx_v.at[0, 0]], rows_a, sem_ga).wait()
    pltpu.make_async_copy(rows_b, acc.at[dst_b], sem_sb).wait()

    plsc.subcore_barrier()

    def _write_blk(i, carry):
        off = pl.multiple_of(sbase + i * 16, 8)
        pltpu.sync_copy(acc.at[pl.ds(off, 16)], out_hbm.at[c, pl.ds(off, 16)])
        return carry

    lax.fori_loop(0, STRIPE // 16, _write_blk, 0)

    @pl.when(s == NS - 1)
    def _():
        pltpu.sync_copy(acc.at[pl.ds(NS * STRIPE, 16)],
                        out_hbm.at[c, pl.ds(NS * STRIPE, 16)])


def kernel(h, edge_index, etype, norm, basis1, coeff1, bias1,
           basis2, coeff2, bias2):
    pad = EP_ROWS * CHE - E
    padi = jnp.zeros((pad,), edge_index.dtype)
    src2 = jnp.concatenate([edge_index[0], padi]).reshape(EP_ROWS, CHE)
    dst2 = jnp.concatenate([edge_index[1], padi]).reshape(EP_ROWS, CHE)
    et2 = jnp.concatenate([etype, padi]).reshape(EP_ROWS, CHE)
    norm2 = jnp.concatenate(
        [norm.reshape(E), jnp.zeros((pad,), norm.dtype)]).reshape(EP_ROWS, CHE)
    b1 = bias1.reshape(1, D)
    b2 = bias2.reshape(1, D)

    gidx2 = _pack_gidx(src2, et2)
    proj1 = _proj1(coeff1, basis1, h).reshape(N * R, D)
    parts1 = _sc_edge_pass(proj1, gidx2, dst2, norm2)
    proj2 = _proj2(coeff2, basis2, parts1, b1).reshape(N * R, D)
    parts2 = _sc_edge_pass(proj2, gidx2, dst2, norm2)
    return _merge(parts2, b2)
